# Initial kernel scaffold; baseline (speedup 1.0000x reference)
import jax
import jax.numpy as jnp
from jax.experimental import pallas as pl


def kernel(bin_center, ground_truth):
    raise NotImplementedError("write your pallas kernel here")


# speedup vs baseline: 6.0520x; 6.0520x over previous
import functools

import jax
import jax.numpy as jnp
from jax import lax
from jax.experimental import pallas as pl
from jax.experimental.pallas import tpu as pltpu
from jax.experimental.pallas import tpu_sc as plsc

B = 4
N = 256
NBPAD = 384
P = 147456
NTILES = 32
CHUNK = P // NTILES
NVREG = CHUNK // 16
NSEG = 272
SENT = 1e9
PADV = 2e9
NEG = -1e9
THRESH = 0.001


def _sort_bins_body(bins_ref, out_ref):
    for b in range(B):
        s = bins_ref[b, :]
        col = s[:, None]
        row = s[None, :]
        ii = lax.broadcasted_iota(jnp.int32, (N, N), 0)
        jj = lax.broadcasted_iota(jnp.int32, (N, N), 1)
        less = (row < col) | ((row == col) & (jj < ii))
        rank = jnp.sum(less.astype(jnp.float32), axis=1)
        onehot = rank[:, None] == jj.astype(jnp.float32)
        sorted_s = jnp.sum(jnp.where(onehot, col, 0.0), axis=0)
        out_ref[b, 0:N] = sorted_s
        out_ref[b, N:NBPAD] = jnp.full((NBPAD - N,), PADV, jnp.float32)


def _sc_body(gt_hbm, bins_hbm, segmin_hbm, segmax_hbm, bwd_hbm, cnt_hbm,
             gt_v, bins_v, segmin_v, segmax_v, bwd_v, cnt_v):
    wid = lax.axis_index("s") * 2 + lax.axis_index("c")
    base = wid * CHUNK
    pltpu.sync_copy(bins_hbm, bins_v)
    for b in range(B):
        pltpu.sync_copy(gt_hbm.at[b, pl.ds(base, CHUNK)], gt_v.at[b])

    def init_body(j, carry):
        for b in range(B):
            segmin_v[b, j] = jnp.full((16,), SENT, jnp.float32)
            segmax_v[b, j] = jnp.full((16,), NEG, jnp.float32)
        return carry

    lax.fori_loop(0, NSEG, init_body, 0)

    lane = lax.iota(jnp.int32, 16)
    for b in range(B):
        bvec = jnp.full((16,), b, jnp.int32)

        def body(j, carry, b=b, bvec=bvec):
            bwd_acc, cnt_acc = carry
            v = gt_v[b, pl.ds(j * 16, 16)]
            mask = v >= THRESH
            veff = jnp.where(mask, v, SENT)
            idx = jnp.zeros((16,), jnp.int32)
            for w in (256, 128, 64, 32, 16, 8, 4, 2, 1):
                x = plsc.load_gather(bins_v, [bvec, idx + (w - 1)])
                idx = idx + jnp.where(x <= veff, w, 0)
            lo = jnp.maximum(idx - 1, 0)
            hi = jnp.minimum(idx, N - 1)
            a = plsc.load_gather(bins_v, [bvec, lo])
            c = plsc.load_gather(bins_v, [bvec, hi])
            da = veff - a
            dc = c - veff
            d = jnp.minimum(da * da, dc * dc)
            bwd_acc = bwd_acc + jnp.where(mask, d, 0.0)
            cnt_acc = cnt_acc + jnp.where(mask, 1.0, 0.0)
            cur = plsc.load_gather(segmin_v, [bvec, idx, lane])
            plsc.store_scatter(segmin_v, [bvec, idx, lane],
                               jnp.minimum(cur, veff))
            curx = plsc.load_gather(segmax_v, [bvec, idx, lane])
            plsc.store_scatter(segmax_v, [bvec, idx, lane],
                               jnp.maximum(curx, jnp.where(mask, veff, NEG)))
            return bwd_acc, cnt_acc

        zero = jnp.zeros((16,), jnp.float32)
        bwd_acc, cnt_acc = lax.fori_loop(0, NVREG, body, (zero, zero))
        bwd_v[b] = bwd_acc
        cnt_v[b] = cnt_acc

    pltpu.sync_copy(segmin_v, segmin_hbm.at[wid])
    pltpu.sync_copy(segmax_v, segmax_hbm.at[wid])
    pltpu.sync_copy(bwd_v, bwd_hbm.at[wid])
    pltpu.sync_copy(cnt_v, cnt_hbm.at[wid])


@functools.lru_cache(maxsize=None)
def _build_sc_chamfer():
    return functools.partial(
        pl.kernel,
        out_type=(
            jax.ShapeDtypeStruct((NTILES, B, NSEG, 16), jnp.float32),
            jax.ShapeDtypeStruct((NTILES, B, NSEG, 16), jnp.float32),
            jax.ShapeDtypeStruct((NTILES, B, 16), jnp.float32),
            jax.ShapeDtypeStruct((NTILES, B, 16), jnp.float32),
        ),
        mesh=plsc.VectorSubcoreMesh(core_axis_name="c", subcore_axis_name="s",
                                    num_cores=2, num_subcores=16),
        compiler_params=pltpu.CompilerParams(use_tc_tiling_on_sc=False,
                                             needs_layout_passes=False),
        scratch_types=[
            pltpu.VMEM((B, CHUNK), jnp.float32),
            pltpu.VMEM((B, NBPAD), jnp.float32),
            pltpu.VMEM((B, NSEG, 16), jnp.float32),
            pltpu.VMEM((B, NSEG, 16), jnp.float32),
            pltpu.VMEM((B, 16), jnp.float32),
            pltpu.VMEM((B, 16), jnp.float32),
        ],
    )(_sc_body)


def _tail_body(bins_ref, segmin_ref, segmax_ref, bwd_ref, cnt_ref, out_ref):
    lengths = [jnp.sum(cnt_ref[b]) for b in range(B)]
    max_len = jnp.maximum(jnp.maximum(lengths[0], lengths[1]),
                          jnp.maximum(lengths[2], lengths[3]))
    total = jnp.float32(0.0)
    for b in range(B):
        sm = jnp.min(segmin_ref[b], axis=1)
        sx = jnp.max(segmax_ref[b], axis=1)
        kk = lax.broadcasted_iota(jnp.int32, (N, NSEG), 1)
        nn = lax.broadcasted_iota(jnp.int32, (N, NSEG), 0)
        below = jnp.max(jnp.where(kk <= nn, sx[None, :], NEG), axis=1)
        above = jnp.min(jnp.where(kk > nn, sm[None, :], SENT), axis=1)
        s = bins_ref[b, 0:N]
        d1 = s - below
        d2 = above - s
        fwd = jnp.minimum(d1 * d1, d2 * d2)
        pad = max_len - lengths[b]
        s2 = s * s
        fwd = jnp.where(pad > 0, jnp.minimum(fwd, s2), fwd)
        total = total + jnp.sum(fwd) + jnp.sum(bwd_ref[b]) + pad * jnp.min(s2)
    out_ref[0, 0] = total / B


def kernel(bin_center, ground_truth):
    bins2 = jnp.reshape(bin_center, (B, N))
    gt = jnp.reshape(ground_truth, (B, P))
    bins_sorted = pl.pallas_call(
        _sort_bins_body,
        out_shape=jax.ShapeDtypeStruct((B, NBPAD), jnp.float32),
    )(bins2)
    segmin_p, segmax_p, bwd_p, cnt_p = _build_sc_chamfer()(gt, bins_sorted)
    segmin_t = jnp.reshape(jnp.transpose(segmin_p, (1, 2, 0, 3)),
                           (B, NSEG, NTILES * 16))
    segmax_t = jnp.reshape(jnp.transpose(segmax_p, (1, 2, 0, 3)),
                           (B, NSEG, NTILES * 16))
    bwd_t = jnp.reshape(jnp.transpose(bwd_p, (1, 0, 2)), (B, NTILES * 16))
    cnt_t = jnp.reshape(jnp.transpose(cnt_p, (1, 0, 2)), (B, NTILES * 16))
    loss = pl.pallas_call(
        _tail_body,
        out_shape=jax.ShapeDtypeStruct((1, 1), jnp.float32),
        out_specs=pl.BlockSpec(memory_space=pltpu.SMEM),
    )(bins_sorted, segmin_t, segmax_t, bwd_t, cnt_t)
    return jnp.reshape(loss, ())


# speedup vs baseline: 6.4320x; 1.0628x over previous
import functools

import jax
import jax.numpy as jnp
from jax import lax
from jax.experimental import pallas as pl
from jax.experimental.pallas import tpu as pltpu
from jax.experimental.pallas import tpu_sc as plsc

B = 4
N = 256
NBPAD = 384
P = 147456
NTILES = 32
CHUNK = P // NTILES
NVREG = CHUNK // 16
NSEG = 272
SENT = 1e9
PADV = 2e9
NEG = -1e9
THRESH = 0.001


def _sort_bins_body(bins_ref, out_ref):
    for b in range(B):
        s = bins_ref[b, :]
        col = s[:, None]
        row = s[None, :]
        ii = lax.broadcasted_iota(jnp.int32, (N, N), 0)
        jj = lax.broadcasted_iota(jnp.int32, (N, N), 1)
        less = (row < col) | ((row == col) & (jj < ii))
        rank = jnp.sum(less.astype(jnp.float32), axis=1)
        onehot = rank[:, None] == jj.astype(jnp.float32)
        sorted_s = jnp.sum(jnp.where(onehot, col, 0.0), axis=0)
        out_ref[b, 0:N] = sorted_s
        out_ref[b, N:NBPAD] = jnp.full((NBPAD - N,), PADV, jnp.float32)


UNROLL = 4


def _sc_body(gt_hbm, bins_hbm, segmin_hbm, segmax_hbm, bwd_hbm, cnt_hbm,
             gt_v, bins_v, segmin_v, segmax_v, bwd_v, cnt_v):
    wid = lax.axis_index("s") * 2 + lax.axis_index("c")
    base = wid * CHUNK
    pltpu.sync_copy(bins_hbm, bins_v)
    for b in range(B):
        pltpu.sync_copy(gt_hbm.at[b, pl.ds(base, CHUNK)], gt_v.at[b])

    def init_body(j, carry):
        segmin_v[pl.ds(j * 16, 16)] = jnp.full((16,), SENT, jnp.float32)
        segmax_v[pl.ds(j * 16, 16)] = jnp.full((16,), NEG, jnp.float32)
        return carry

    lax.fori_loop(0, B * NSEG, init_body, 0)

    lane = lax.iota(jnp.int32, 16)
    for b in range(B):
        bofs = b * NBPAD
        seg_base = lane + b * NSEG * 16

        def body(j, carry, b=b, bofs=bofs, seg_base=seg_base):
            bwd_acc, cnt_acc = carry
            for u in range(UNROLL):
                v = gt_v[b, pl.ds((j * UNROLL + u) * 16, 16)]
                mask = v >= THRESH
                veff = jnp.where(mask, v, SENT)
                idx = jnp.zeros((16,), jnp.int32)
                for w in (256, 128, 64, 32, 16, 8, 4, 2, 1):
                    x = plsc.load_gather(bins_v, [idx + (w - 1 + bofs)])
                    idx = idx + jnp.where(x <= veff, w, 0)
                lo = jnp.maximum(idx - 1, 0) + bofs
                hi = jnp.minimum(idx, N - 1) + bofs
                a = plsc.load_gather(bins_v, [lo])
                c = plsc.load_gather(bins_v, [hi])
                da = veff - a
                dc = c - veff
                d = jnp.minimum(da * da, dc * dc)
                bwd_acc = bwd_acc + jnp.where(mask, d, 0.0)
                cnt_acc = cnt_acc + jnp.where(mask, 1.0, 0.0)
                fidx = (idx << 4) + seg_base
                cur = plsc.load_gather(segmin_v, [fidx])
                plsc.store_scatter(segmin_v, [fidx], jnp.minimum(cur, veff))
                curx = plsc.load_gather(segmax_v, [fidx])
                plsc.store_scatter(segmax_v, [fidx],
                                   jnp.maximum(curx, jnp.where(mask, veff, NEG)))
            return bwd_acc, cnt_acc

        zero = jnp.zeros((16,), jnp.float32)
        bwd_acc, cnt_acc = lax.fori_loop(0, NVREG // UNROLL, body, (zero, zero))
        bwd_v[b] = bwd_acc
        cnt_v[b] = cnt_acc

    pltpu.sync_copy(segmin_v, segmin_hbm.at[wid])
    pltpu.sync_copy(segmax_v, segmax_hbm.at[wid])
    pltpu.sync_copy(bwd_v, bwd_hbm.at[wid])
    pltpu.sync_copy(cnt_v, cnt_hbm.at[wid])


@functools.lru_cache(maxsize=None)
def _build_sc_chamfer():
    return functools.partial(
        pl.kernel,
        out_type=(
            jax.ShapeDtypeStruct((NTILES, B * NSEG * 16), jnp.float32),
            jax.ShapeDtypeStruct((NTILES, B * NSEG * 16), jnp.float32),
            jax.ShapeDtypeStruct((NTILES, B, 16), jnp.float32),
            jax.ShapeDtypeStruct((NTILES, B, 16), jnp.float32),
        ),
        mesh=plsc.VectorSubcoreMesh(core_axis_name="c", subcore_axis_name="s",
                                    num_cores=2, num_subcores=16),
        compiler_params=pltpu.CompilerParams(use_tc_tiling_on_sc=False,
                                             needs_layout_passes=False),
        scratch_types=[
            pltpu.VMEM((B, CHUNK), jnp.float32),
            pltpu.VMEM((B * NBPAD,), jnp.float32),
            pltpu.VMEM((B * NSEG * 16,), jnp.float32),
            pltpu.VMEM((B * NSEG * 16,), jnp.float32),
            pltpu.VMEM((B, 16), jnp.float32),
            pltpu.VMEM((B, 16), jnp.float32),
        ],
    )(_sc_body)


def _tail_body(bins_ref, segmin_ref, segmax_ref, bwd_ref, cnt_ref, out_ref):
    lengths = [jnp.sum(cnt_ref[b]) for b in range(B)]
    max_len = jnp.maximum(jnp.maximum(lengths[0], lengths[1]),
                          jnp.maximum(lengths[2], lengths[3]))
    total = jnp.float32(0.0)
    for b in range(B):
        sm = jnp.min(segmin_ref[b], axis=1)
        sx = jnp.max(segmax_ref[b], axis=1)
        kk = lax.broadcasted_iota(jnp.int32, (N, NSEG), 1)
        nn = lax.broadcasted_iota(jnp.int32, (N, NSEG), 0)
        below = jnp.max(jnp.where(kk <= nn, sx[None, :], NEG), axis=1)
        above = jnp.min(jnp.where(kk > nn, sm[None, :], SENT), axis=1)
        s = bins_ref[b, 0:N]
        d1 = s - below
        d2 = above - s
        fwd = jnp.minimum(d1 * d1, d2 * d2)
        pad = max_len - lengths[b]
        s2 = s * s
        fwd = jnp.where(pad > 0, jnp.minimum(fwd, s2), fwd)
        total = total + jnp.sum(fwd) + jnp.sum(bwd_ref[b]) + pad * jnp.min(s2)
    out_ref[0, 0] = total / B


def kernel(bin_center, ground_truth):
    bins2 = jnp.reshape(bin_center, (B, N))
    gt = jnp.reshape(ground_truth, (B, P))
    bins_sorted = pl.pallas_call(
        _sort_bins_body,
        out_shape=jax.ShapeDtypeStruct((B, NBPAD), jnp.float32),
    )(bins2)
    segmin_p, segmax_p, bwd_p, cnt_p = _build_sc_chamfer()(
        gt, jnp.reshape(bins_sorted, (B * NBPAD,)))
    segmin_p = jnp.reshape(segmin_p, (NTILES, B, NSEG, 16))
    segmax_p = jnp.reshape(segmax_p, (NTILES, B, NSEG, 16))
    segmin_t = jnp.reshape(jnp.transpose(segmin_p, (1, 2, 0, 3)),
                           (B, NSEG, NTILES * 16))
    segmax_t = jnp.reshape(jnp.transpose(segmax_p, (1, 2, 0, 3)),
                           (B, NSEG, NTILES * 16))
    bwd_t = jnp.reshape(jnp.transpose(bwd_p, (1, 0, 2)), (B, NTILES * 16))
    cnt_t = jnp.reshape(jnp.transpose(cnt_p, (1, 0, 2)), (B, NTILES * 16))
    loss = pl.pallas_call(
        _tail_body,
        out_shape=jax.ShapeDtypeStruct((1, 1), jnp.float32),
        out_specs=pl.BlockSpec(memory_space=pltpu.SMEM),
    )(bins_sorted, segmin_t, segmax_t, bwd_t, cnt_t)
    return jnp.reshape(loss, ())


# speedup vs baseline: 9.2252x; 1.4343x over previous
import functools

import jax
import jax.numpy as jnp
from jax import lax
from jax.experimental import pallas as pl
from jax.experimental.pallas import tpu as pltpu
from jax.experimental.pallas import tpu_sc as plsc

B = 4
N = 256
NBPAD = 384
P = 147456
NTILES = 32
CHUNK = P // NTILES
NVREG = CHUNK // 16
NSEG = 272
SENT = 1e9
PADV = 2e9
NEG = -1e9
THRESH = 0.001


def _sort_bins_body(bins_ref, out_ref):
    for b in range(B):
        s = bins_ref[b, :]
        col = s[:, None]
        row = s[None, :]
        ii = lax.broadcasted_iota(jnp.int32, (N, N), 0)
        jj = lax.broadcasted_iota(jnp.int32, (N, N), 1)
        less = (row < col) | ((row == col) & (jj < ii))
        rank = jnp.sum(less.astype(jnp.float32), axis=1)
        onehot = rank[:, None] == jj.astype(jnp.float32)
        sorted_s = jnp.sum(jnp.where(onehot, col, 0.0), axis=0)
        out_ref[b, 0:N] = sorted_s
        out_ref[b, N:NBPAD] = jnp.full((NBPAD - N,), PADV, jnp.float32)


UNROLL = 4


def _sc_body(gt_hbm, bins_hbm, segmin_hbm, segmax_hbm, bwd_hbm, cnt_hbm,
             gt_v, bins_v, segmin_v, segmax_v, bwd_v, cnt_v):
    wid = lax.axis_index("s") * 2 + lax.axis_index("c")
    base = wid * CHUNK
    pltpu.sync_copy(bins_hbm, bins_v)
    for b in range(B):
        pltpu.sync_copy(gt_hbm.at[b, pl.ds(base, CHUNK)], gt_v.at[b])

    def init_body(j, carry):
        segmin_v[pl.ds(j * 16, 16)] = jnp.full((16,), SENT, jnp.float32)
        segmax_v[pl.ds(j * 16, 16)] = jnp.full((16,), NEG, jnp.float32)
        return carry

    lax.fori_loop(0, B * NSEG, init_body, 0)

    lane = lax.iota(jnp.int32, 16)
    for b in range(B):
        bofs = b * NBPAD
        seg_base = lane + b * NSEG * 16

        def body(j, carry, b=b, bofs=bofs, seg_base=seg_base):
            bwd_acc, cnt_acc = carry
            masks, veffs = [], []
            idxs = []
            for u in range(UNROLL):
                v = gt_v[b, pl.ds((j * UNROLL + u) * 16, 16)]
                mask = v >= THRESH
                masks.append(mask)
                veffs.append(jnp.where(mask, v, SENT))
                idxs.append(jnp.zeros((16,), jnp.int32))
            for w in (256, 128, 64, 32, 16, 8, 4, 2, 1):
                for u in range(UNROLL):
                    x = plsc.load_gather(bins_v, [idxs[u] + (w - 1 + bofs)])
                    idxs[u] = idxs[u] + jnp.where(x <= veffs[u], w, 0)
            nears = []
            for u in range(UNROLL):
                lo = jnp.maximum(idxs[u] - 1, 0) + bofs
                hi = jnp.minimum(idxs[u], N - 1) + bofs
                nears.append((plsc.load_gather(bins_v, [lo]),
                              plsc.load_gather(bins_v, [hi])))
            for u in range(UNROLL):
                a, c = nears[u]
                da = veffs[u] - a
                dc = c - veffs[u]
                d = jnp.minimum(da * da, dc * dc)
                bwd_acc = bwd_acc + jnp.where(masks[u], d, 0.0)
                cnt_acc = cnt_acc + jnp.where(masks[u], 1.0, 0.0)
            fidxs = [(idxs[u] << 4) + seg_base for u in range(UNROLL)]
            for u in range(UNROLL):
                cur = plsc.load_gather(segmin_v, [fidxs[u]])
                plsc.store_scatter(segmin_v, [fidxs[u]],
                                   jnp.minimum(cur, veffs[u]))
                curx = plsc.load_gather(segmax_v, [fidxs[u]])
                plsc.store_scatter(segmax_v, [fidxs[u]],
                                   jnp.maximum(curx,
                                               jnp.where(masks[u], veffs[u],
                                                         NEG)))
            return bwd_acc, cnt_acc

        zero = jnp.zeros((16,), jnp.float32)
        bwd_acc, cnt_acc = lax.fori_loop(0, NVREG // UNROLL, body, (zero, zero))
        bwd_v[b] = bwd_acc
        cnt_v[b] = cnt_acc

    pltpu.sync_copy(segmin_v, segmin_hbm.at[wid])
    pltpu.sync_copy(segmax_v, segmax_hbm.at[wid])
    pltpu.sync_copy(bwd_v, bwd_hbm.at[wid])
    pltpu.sync_copy(cnt_v, cnt_hbm.at[wid])


@functools.lru_cache(maxsize=None)
def _build_sc_chamfer():
    return functools.partial(
        pl.kernel,
        out_type=(
            jax.ShapeDtypeStruct((NTILES, B * NSEG * 16), jnp.float32),
            jax.ShapeDtypeStruct((NTILES, B * NSEG * 16), jnp.float32),
            jax.ShapeDtypeStruct((NTILES, B, 16), jnp.float32),
            jax.ShapeDtypeStruct((NTILES, B, 16), jnp.float32),
        ),
        mesh=plsc.VectorSubcoreMesh(core_axis_name="c", subcore_axis_name="s",
                                    num_cores=2, num_subcores=16),
        compiler_params=pltpu.CompilerParams(use_tc_tiling_on_sc=False,
                                             needs_layout_passes=False),
        scratch_types=[
            pltpu.VMEM((B, CHUNK), jnp.float32),
            pltpu.VMEM((B * NBPAD,), jnp.float32),
            pltpu.VMEM((B * NSEG * 16,), jnp.float32),
            pltpu.VMEM((B * NSEG * 16,), jnp.float32),
            pltpu.VMEM((B, 16), jnp.float32),
            pltpu.VMEM((B, 16), jnp.float32),
        ],
    )(_sc_body)


def _tail_body(bins_ref, segmin_ref, segmax_ref, bwd_ref, cnt_ref, out_ref):
    lengths = [jnp.sum(cnt_ref[b]) for b in range(B)]
    max_len = jnp.maximum(jnp.maximum(lengths[0], lengths[1]),
                          jnp.maximum(lengths[2], lengths[3]))
    total = jnp.float32(0.0)
    for b in range(B):
        sm = jnp.min(segmin_ref[b], axis=1)
        sx = jnp.max(segmax_ref[b], axis=1)
        kk = lax.broadcasted_iota(jnp.int32, (N, NSEG), 1)
        nn = lax.broadcasted_iota(jnp.int32, (N, NSEG), 0)
        below = jnp.max(jnp.where(kk <= nn, sx[None, :], NEG), axis=1)
        above = jnp.min(jnp.where(kk > nn, sm[None, :], SENT), axis=1)
        s = bins_ref[b, 0:N]
        d1 = s - below
        d2 = above - s
        fwd = jnp.minimum(d1 * d1, d2 * d2)
        pad = max_len - lengths[b]
        s2 = s * s
        fwd = jnp.where(pad > 0, jnp.minimum(fwd, s2), fwd)
        total = total + jnp.sum(fwd) + jnp.sum(bwd_ref[b]) + pad * jnp.min(s2)
    out_ref[0, 0] = total / B


def kernel(bin_center, ground_truth):
    bins2 = jnp.reshape(bin_center, (B, N))
    gt = jnp.reshape(ground_truth, (B, P))
    bins_sorted = pl.pallas_call(
        _sort_bins_body,
        out_shape=jax.ShapeDtypeStruct((B, NBPAD), jnp.float32),
    )(bins2)
    segmin_p, segmax_p, bwd_p, cnt_p = _build_sc_chamfer()(
        gt, jnp.reshape(bins_sorted, (B * NBPAD,)))
    segmin_p = jnp.reshape(segmin_p, (NTILES, B, NSEG, 16))
    segmax_p = jnp.reshape(segmax_p, (NTILES, B, NSEG, 16))
    segmin_t = jnp.reshape(jnp.transpose(segmin_p, (1, 2, 0, 3)),
                           (B, NSEG, NTILES * 16))
    segmax_t = jnp.reshape(jnp.transpose(segmax_p, (1, 2, 0, 3)),
                           (B, NSEG, NTILES * 16))
    bwd_t = jnp.reshape(jnp.transpose(bwd_p, (1, 0, 2)), (B, NTILES * 16))
    cnt_t = jnp.reshape(jnp.transpose(cnt_p, (1, 0, 2)), (B, NTILES * 16))
    loss = pl.pallas_call(
        _tail_body,
        out_shape=jax.ShapeDtypeStruct((1, 1), jnp.float32),
        out_specs=pl.BlockSpec(memory_space=pltpu.SMEM),
    )(bins_sorted, segmin_t, segmax_t, bwd_t, cnt_t)
    return jnp.reshape(loss, ())


# speedup vs baseline: 10.0595x; 1.0904x over previous
import functools

import jax
import jax.numpy as jnp
from jax import lax
from jax.experimental import pallas as pl
from jax.experimental.pallas import tpu as pltpu
from jax.experimental.pallas import tpu_sc as plsc

B = 4
N = 256
NBPAD = 384
P = 147456
NTILES = 32
CHUNK = P // NTILES
NVREG = CHUNK // 16
NSEG = 272
SENT = 1e9
PADV = 2e9
NEG = -1e9
THRESH = 0.001


def _sort_bins_body(bins_ref, out_ref):
    for b in range(B):
        s = bins_ref[b, :]
        col = s[:, None]
        row = s[None, :]
        ii = lax.broadcasted_iota(jnp.int32, (N, N), 0)
        jj = lax.broadcasted_iota(jnp.int32, (N, N), 1)
        less = (row < col) | ((row == col) & (jj < ii))
        rank = jnp.sum(less.astype(jnp.float32), axis=1)
        onehot = rank[:, None] == jj.astype(jnp.float32)
        sorted_s = jnp.sum(jnp.where(onehot, col, 0.0), axis=0)
        out_ref[b, 0:N] = sorted_s
        out_ref[b, N:NBPAD] = jnp.full((NBPAD - N,), PADV, jnp.float32)


UNROLL = 8


def _sc_body(gt_hbm, bins_hbm, segmin_hbm, segmax_hbm, bwd_hbm, cnt_hbm,
             gt_v, bins_v, segmin_v, segmax_v, bwd_v, cnt_v):
    wid = lax.axis_index("s") * 2 + lax.axis_index("c")
    base = wid * CHUNK
    pltpu.sync_copy(bins_hbm, bins_v)
    for b in range(B):
        pltpu.sync_copy(gt_hbm.at[b, pl.ds(base, CHUNK)], gt_v.at[b])

    def init_body(j, carry):
        segmin_v[pl.ds(j * 16, 16)] = jnp.full((16,), SENT, jnp.float32)
        segmax_v[pl.ds(j * 16, 16)] = jnp.full((16,), NEG, jnp.float32)
        return carry

    lax.fori_loop(0, B * NSEG, init_body, 0)

    lane = lax.iota(jnp.int32, 16)
    for b in range(B):
        bofs = b * NBPAD
        seg_base = lane + b * NSEG * 16

        def body(j, carry, b=b, bofs=bofs, seg_base=seg_base):
            bwd_acc, cnt_acc = carry
            masks, veffs = [], []
            idxs = []
            for u in range(UNROLL):
                v = gt_v[b, pl.ds((j * UNROLL + u) * 16, 16)]
                mask = v >= THRESH
                masks.append(mask)
                veffs.append(jnp.where(mask, v, SENT))
                idxs.append(jnp.zeros((16,), jnp.int32))
            for w in (256, 128, 64, 32, 16, 8, 4, 2, 1):
                for u in range(UNROLL):
                    x = plsc.load_gather(bins_v, [idxs[u] + (w - 1 + bofs)])
                    idxs[u] = idxs[u] + jnp.where(x <= veffs[u], w, 0)
            nears = []
            for u in range(UNROLL):
                lo = jnp.maximum(idxs[u] - 1, 0) + bofs
                hi = jnp.minimum(idxs[u], N - 1) + bofs
                nears.append((plsc.load_gather(bins_v, [lo]),
                              plsc.load_gather(bins_v, [hi])))
            for u in range(UNROLL):
                a, c = nears[u]
                da = veffs[u] - a
                dc = c - veffs[u]
                d = jnp.minimum(da * da, dc * dc)
                bwd_acc = bwd_acc + jnp.where(masks[u], d, 0.0)
                cnt_acc = cnt_acc + jnp.where(masks[u], 1.0, 0.0)
            fidxs = [(idxs[u] << 4) + seg_base for u in range(UNROLL)]
            for u in range(UNROLL):
                cur = plsc.load_gather(segmin_v, [fidxs[u]])
                plsc.store_scatter(segmin_v, [fidxs[u]],
                                   jnp.minimum(cur, veffs[u]))
                curx = plsc.load_gather(segmax_v, [fidxs[u]])
                plsc.store_scatter(segmax_v, [fidxs[u]],
                                   jnp.maximum(curx,
                                               jnp.where(masks[u], veffs[u],
                                                         NEG)))
            return bwd_acc, cnt_acc

        zero = jnp.zeros((16,), jnp.float32)
        bwd_acc, cnt_acc = lax.fori_loop(0, NVREG // UNROLL, body, (zero, zero))
        bwd_v[b] = bwd_acc
        cnt_v[b] = cnt_acc

    pltpu.sync_copy(segmin_v, segmin_hbm.at[wid])
    pltpu.sync_copy(segmax_v, segmax_hbm.at[wid])
    pltpu.sync_copy(bwd_v, bwd_hbm.at[wid])
    pltpu.sync_copy(cnt_v, cnt_hbm.at[wid])


@functools.lru_cache(maxsize=None)
def _build_sc_chamfer():
    return functools.partial(
        pl.kernel,
        out_type=(
            jax.ShapeDtypeStruct((NTILES, B * NSEG * 16), jnp.float32),
            jax.ShapeDtypeStruct((NTILES, B * NSEG * 16), jnp.float32),
            jax.ShapeDtypeStruct((NTILES, B, 16), jnp.float32),
            jax.ShapeDtypeStruct((NTILES, B, 16), jnp.float32),
        ),
        mesh=plsc.VectorSubcoreMesh(core_axis_name="c", subcore_axis_name="s",
                                    num_cores=2, num_subcores=16),
        compiler_params=pltpu.CompilerParams(use_tc_tiling_on_sc=False,
                                             needs_layout_passes=False),
        scratch_types=[
            pltpu.VMEM((B, CHUNK), jnp.float32),
            pltpu.VMEM((B * NBPAD,), jnp.float32),
            pltpu.VMEM((B * NSEG * 16,), jnp.float32),
            pltpu.VMEM((B * NSEG * 16,), jnp.float32),
            pltpu.VMEM((B, 16), jnp.float32),
            pltpu.VMEM((B, 16), jnp.float32),
        ],
    )(_sc_body)


def _tail_body(bins_ref, segmin_ref, segmax_ref, bwd_ref, cnt_ref, out_ref):
    lengths = [jnp.sum(cnt_ref[b]) for b in range(B)]
    max_len = jnp.maximum(jnp.maximum(lengths[0], lengths[1]),
                          jnp.maximum(lengths[2], lengths[3]))
    total = jnp.float32(0.0)
    for b in range(B):
        sm = jnp.min(segmin_ref[b], axis=1)
        sx = jnp.max(segmax_ref[b], axis=1)
        kk = lax.broadcasted_iota(jnp.int32, (N, NSEG), 1)
        nn = lax.broadcasted_iota(jnp.int32, (N, NSEG), 0)
        below = jnp.max(jnp.where(kk <= nn, sx[None, :], NEG), axis=1)
        above = jnp.min(jnp.where(kk > nn, sm[None, :], SENT), axis=1)
        s = bins_ref[b, 0:N]
        d1 = s - below
        d2 = above - s
        fwd = jnp.minimum(d1 * d1, d2 * d2)
        pad = max_len - lengths[b]
        s2 = s * s
        fwd = jnp.where(pad > 0, jnp.minimum(fwd, s2), fwd)
        total = total + jnp.sum(fwd) + jnp.sum(bwd_ref[b]) + pad * jnp.min(s2)
    out_ref[0, 0] = total / B


def kernel(bin_center, ground_truth):
    bins2 = jnp.reshape(bin_center, (B, N))
    gt = jnp.reshape(ground_truth, (B, P))
    bins_sorted = pl.pallas_call(
        _sort_bins_body,
        out_shape=jax.ShapeDtypeStruct((B, NBPAD), jnp.float32),
    )(bins2)
    segmin_p, segmax_p, bwd_p, cnt_p = _build_sc_chamfer()(
        gt, jnp.reshape(bins_sorted, (B * NBPAD,)))
    segmin_p = jnp.reshape(segmin_p, (NTILES, B, NSEG, 16))
    segmax_p = jnp.reshape(segmax_p, (NTILES, B, NSEG, 16))
    segmin_t = jnp.reshape(jnp.transpose(segmin_p, (1, 2, 0, 3)),
                           (B, NSEG, NTILES * 16))
    segmax_t = jnp.reshape(jnp.transpose(segmax_p, (1, 2, 0, 3)),
                           (B, NSEG, NTILES * 16))
    bwd_t = jnp.reshape(jnp.transpose(bwd_p, (1, 0, 2)), (B, NTILES * 16))
    cnt_t = jnp.reshape(jnp.transpose(cnt_p, (1, 0, 2)), (B, NTILES * 16))
    loss = pl.pallas_call(
        _tail_body,
        out_shape=jax.ShapeDtypeStruct((1, 1), jnp.float32),
        out_specs=pl.BlockSpec(memory_space=pltpu.SMEM),
    )(bins_sorted, segmin_t, segmax_t, bwd_t, cnt_t)
    return jnp.reshape(loss, ())
